# baseline (speedup 1.0000x reference)
import jax
import jax.numpy as jnp
import numpy as np
from jax.experimental import pallas as pl
from jax.experimental.pallas import tpu as pltpu

_N = 8192
_F = 32
_K = 128
_R = 64
_INT_MIN = np.int32(-2147483648)
_INT_MAX = np.int32(2147483647)


def _main_kernel(big, W1a, W1b, b1r, W2T, b2c, W3c,
                 Rg, tg, b3, loss_ref, R_ref, t_ref):
    shp = (_R, 128)
    hs3 = lambda c: big[c]
    ht3 = lambda c: big[_F + c]
    xs3 = lambda c: big[2 * _F + c]
    xt3 = lambda c: big[2 * _F + 3 + c]
    labv = big[2 * _F + 6]

    sim = jnp.zeros(shp, jnp.float32)
    ns2 = jnp.zeros(shp, jnp.float32)
    nt2 = jnp.zeros(shp, jnp.float32)
    for c in range(_F):
        a = hs3(c)
        b = ht3(c)
        sim = sim + a * b
        ns2 = ns2 + a * a
        nt2 = nt2 + b * b

    ch = jnp.zeros(shp, jnp.float32)
    xsr = [xs3(c) for c in range(3)]
    xtr = [xt3(c) for c in range(3)]
    for c in range(3):
        xtf_c = (Rg[c, 0] * xsr[0] + Rg[c, 1] * xsr[1]
                 + Rg[c, 2] * xsr[2] + tg[c, 0])
        d = xtf_c - xtr[c]
        ch = ch + d * d
    rot_loss = jnp.sum(ch * labv) / _N

    eps = 1e-8
    den = jnp.maximum(jnp.sqrt(ns2), eps) * jnp.maximum(jnp.sqrt(nt2), eps)
    fdiff = sim / den - labv
    feat_loss = jnp.sum(fdiff * fdiff) / _N
    loss_ref[0, 0] = rot_loss + feat_loss

    bits = jax.lax.bitcast_convert_type(sim, jnp.int32)
    key = bits ^ ((bits >> 31) & jnp.int32(0x7FFFFFFF))
    idx = (jax.lax.broadcasted_iota(jnp.int32, shp, 0) * 128
           + jax.lax.broadcasted_iota(jnp.int32, shp, 1))

    def bs_body(i, c):
        lo, hi = c
        floor_avg = (lo >> 1) + (hi >> 1) + (lo & hi & 1)
        mid = floor_avg + ((lo ^ hi) & 1)
        cnt = jnp.sum((key >= mid).astype(jnp.int32))
        take = cnt >= _K
        return (jnp.where(take, mid, lo),
                jnp.where(take, hi, mid - jnp.int32(1)))

    vK, _ = jax.lax.fori_loop(0, 34, bs_body, (_INT_MIN, _INT_MAX))

    gt_cnt = jnp.sum((key > vK).astype(jnp.int32))
    tslots = _K - gt_cnt
    tie = key == vK

    def ts_body(i, c):
        lo2, hi2 = c
        mid = (lo2 + hi2) // 2
        cnt = jnp.sum((tie & (idx <= mid)).astype(jnp.int32))
        take = cnt >= tslots
        return (jnp.where(take, lo2, mid + jnp.int32(1)),
                jnp.where(take, mid, hi2))

    tstar, _ = jax.lax.fori_loop(0, 14, ts_body,
                                 (jnp.int32(0), jnp.int32(_N - 1)))
    member = (key > vK) | (tie & (idx <= tstar))

    mkey = jnp.max(key)
    i0 = jnp.min(jnp.where(key == mkey, idx, _INT_MAX))

    rowmask = idx == i0
    h1 = b1r[:, :]
    for c in range(_F):
        x_c = jnp.sum(jnp.where(rowmask, hs3(c), 0.0))
        y_c = jnp.sum(jnp.where(rowmask, ht3(c), 0.0))
        h1 = h1 + x_c * W1a[c:c + 1, :] + y_c * W1b[c:c + 1, :]
    h1 = jnp.maximum(h1, 0.0)
    h2 = jnp.sum(W2T[:, :] * h1, axis=1, keepdims=True) + b2c[:, :]
    h2 = jnp.maximum(h2, 0.0)
    pred0 = (jnp.sum(h2 * W3c[:, :]) + b3[0, 0]) / _K

    cond = (pred0 > 0.5) & ((jnp.abs(pred0 - 1.0) < sim) | (pred0 < sim))
    fw0 = jnp.where(member & cond, pred0, sim)
    S = jnp.sum(fw0)
    ws = fw0 / (S + 1e-6)
    m = jnp.max(ws)
    e = jnp.exp(ws - m)
    sm = e / jnp.sum(e)
    w = sm / (jnp.sum(sm) + 1e-6)
    sw = jnp.sum(w)

    wxs = [w * xsr[c] for c in range(3)]
    scv = [jnp.sum(wxs[c]) for c in range(3)]
    tcv = [jnp.sum(w * xtr[d]) for d in range(3)]
    Hm = [[jnp.sum(wxs[c] * xtr[d]) - (2.0 - sw) * scv[c] * tcv[d]
           + (1e-6 if c == d else 0.0) for d in range(3)] for c in range(3)]

    A = [[sum(Hm[k][i] * Hm[k][j] for k in range(3)) for j in range(3)]
         for i in range(3)]
    V = [[jnp.float32(1.0 if i == j else 0.0) for j in range(3)]
         for i in range(3)]
    for _sweep in range(6):
        for (p, q) in ((0, 1), (0, 2), (1, 2)):
            apq = A[p][q]
            skip = jnp.abs(apq) < 1e-36
            tau = (A[q][q] - A[p][p]) / (2.0 * jnp.where(skip, 1.0, apq))
            sgn = jnp.where(tau >= 0.0, 1.0, -1.0)
            t = sgn / (jnp.abs(tau) + jnp.sqrt(1.0 + tau * tau))
            t = jnp.where(skip, 0.0, t)
            cc = 1.0 / jnp.sqrt(1.0 + t * t)
            ss = t * cc
            for k in range(3):
                akp, akq = A[k][p], A[k][q]
                A[k][p] = cc * akp - ss * akq
                A[k][q] = ss * akp + cc * akq
            for k in range(3):
                apk, aqk = A[p][k], A[q][k]
                A[p][k] = cc * apk - ss * aqk
                A[q][k] = ss * apk + cc * aqk
            for k in range(3):
                vkp, vkq = V[k][p], V[k][q]
                V[k][p] = cc * vkp - ss * vkq
                V[k][q] = ss * vkp + cc * vkq
    lam = [A[0][0], A[1][1], A[2][2]]
    cols = [[V[k][i] for k in range(3)] for i in range(3)]
    vdet = jnp.float32(1.0)
    for (i, j) in ((0, 1), (1, 2), (0, 1)):
        swp = lam[i] < lam[j]
        li, lj = lam[i], lam[j]
        lam[i] = jnp.where(swp, lj, li)
        lam[j] = jnp.where(swp, li, lj)
        vdet = jnp.where(swp, -vdet, vdet)
        for k in range(3):
            ci, cj = cols[i][k], cols[j][k]
            cols[i][k] = jnp.where(swp, cj, ci)
            cols[j][k] = jnp.where(swp, ci, cj)
    sig = [jnp.sqrt(jnp.maximum(l, 0.0)) for l in lam]
    Hv = [[sum(Hm[r][k] * cols[i][k] for k in range(3)) for r in range(3)]
          for i in range(2)]
    u1 = [Hv[0][r] / sig[0] for r in range(3)]
    n1 = jnp.sqrt(sum(u * u for u in u1))
    u1 = [u / n1 for u in u1]
    u2 = [Hv[1][r] / sig[1] for r in range(3)]
    n2 = jnp.sqrt(sum(u * u for u in u2))
    u2 = [u / n2 for u in u2]
    u3 = [(u1[1] * u2[2] - u1[2] * u2[1]) * vdet,
          (u1[2] * u2[0] - u1[0] * u2[2]) * vdet,
          (u1[0] * u2[1] - u1[1] * u2[0]) * vdet]
    Rm = [[cols[0][r] * u1[c] + cols[1][r] * u2[c] + cols[2][r] * u3[c]
           for c in range(3)] for r in range(3)]
    for r in range(3):
        for c in range(3):
            R_ref[r, c] = Rm[r][c]
        t_ref[r, 0] = tcv[r] - (Rm[r][0] * scv[0] + Rm[r][1] * scv[1]
                                + Rm[r][2] * scv[2])


def _vm():
    return pl.BlockSpec(memory_space=pltpu.VMEM)


def _sm():
    return pl.BlockSpec(memory_space=pltpu.SMEM)


def kernel(h_src, x_src, edges_src, edge_attr_src, h_tgt, x_tgt, edges_tgt,
           edge_attr_tgt, corr, labels, gt_pose, W1, b1, W2, b2, W3, b3):
    f32 = jnp.float32
    cat = jnp.concatenate(
        [h_src[0], h_tgt[0], x_src[0], x_tgt[0], labels[0][:, None]], axis=1)
    big = jnp.transpose(cat).reshape(2 * _F + 7, _R, 128)
    W1a = W1[:_F, :]
    W1b = W1[_F:, :]
    b1r = b1.reshape(1, _F)
    W2T = jnp.transpose(W2)
    b2c = b2.reshape(_F // 2, 1)
    W3c = W3.reshape(_F // 2, 1)
    Rg = gt_pose[0, :3, :3]
    tg = gt_pose[0, :3, 3:4]
    b3m = b3.reshape(1, 1)

    loss, Rb, tb = pl.pallas_call(
        _main_kernel,
        out_shape=(
            jax.ShapeDtypeStruct((1, 1), f32),
            jax.ShapeDtypeStruct((3, 3), f32),
            jax.ShapeDtypeStruct((3, 1), f32),
        ),
        in_specs=[_vm(),
                  _vm(), _vm(), _vm(), _vm(), _vm(), _vm(),
                  _sm(), _sm(), _sm()],
        out_specs=(_sm(), _sm(), _sm()),
    )(big, W1a, W1b, b1r, W2T, b2c, W3c, Rg, tg, b3m)

    return loss[0, 0], Rb[None], tb[:, 0][None]


# speedup vs baseline: 1.1203x; 1.1203x over previous
import jax
import jax.numpy as jnp
import numpy as np
from jax.experimental import pallas as pl
from jax.experimental.pallas import tpu as pltpu

_N = 8192
_F = 32
_K = 128
_INT_MIN = np.int32(-2147483648)
_INT_MAX = np.int32(2147483647)


def _main_kernel(hsT, htT, xT, lab, W1m, b1r, W2m, b2c, W3c,
                 Rg, tg, b3, loss_ref, R_ref, t_ref):
    hs = hsT[:, :]
    ht = htT[:, :]
    xs = xT[0:3, :]
    xt = xT[3:6, :]
    labv = lab[:, :]
    W1a = W1m[0:_F, :]
    W1b = W1m[_F:2 * _F, :]
    W2T = jnp.transpose(W2m[:, :])

    sim = jnp.sum(hs * ht, axis=0, keepdims=True)

    ch = jnp.zeros((1, _N), jnp.float32)
    for c in range(3):
        xtf_c = (Rg[c, 0] * xs[0:1, :] + Rg[c, 1] * xs[1:2, :]
                 + Rg[c, 2] * xs[2:3, :] + tg[c, 0])
        d = xtf_c - xt[c:c + 1, :]
        ch = ch + d * d
    rot_loss = jnp.sum(ch * labv) / _N

    eps = 1e-8
    ns = jnp.sqrt(jnp.sum(hs * hs, axis=0, keepdims=True))
    nt = jnp.sqrt(jnp.sum(ht * ht, axis=0, keepdims=True))
    den = jnp.maximum(ns, eps) * jnp.maximum(nt, eps)
    fdiff = sim / den - labv
    feat_loss = jnp.sum(fdiff * fdiff) / _N
    loss_ref[0, 0] = rot_loss + feat_loss

    bits = jax.lax.bitcast_convert_type(sim, jnp.int32)
    key = bits ^ ((bits >> 31) & jnp.int32(0x7FFFFFFF))
    lane = jax.lax.broadcasted_iota(jnp.int32, (1, _N), 1)

    def bs_body(i, c):
        lo, hi = c
        floor_avg = (lo >> 1) + (hi >> 1) + (lo & hi & 1)
        mid = floor_avg + ((lo ^ hi) & 1)
        cnt = jnp.sum((key >= mid).astype(jnp.int32))
        take = cnt >= _K
        return (jnp.where(take, mid, lo),
                jnp.where(take, hi, mid - jnp.int32(1)))

    vK, _ = jax.lax.fori_loop(0, 34, bs_body, (_INT_MIN, _INT_MAX))

    gt_cnt = jnp.sum((key > vK).astype(jnp.int32))
    tslots = _K - gt_cnt
    tie = key == vK

    def ts_body(i, c):
        lo2, hi2 = c
        mid = (lo2 + hi2) // 2
        cnt = jnp.sum((tie & (lane <= mid)).astype(jnp.int32))
        take = cnt >= tslots
        return (jnp.where(take, lo2, mid + jnp.int32(1)),
                jnp.where(take, mid, hi2))

    tstar, _ = jax.lax.fori_loop(0, 14, ts_body,
                                 (jnp.int32(0), jnp.int32(_N - 1)))
    member = (key > vK) | (tie & (lane <= tstar))

    mkey = jnp.max(key)
    i0 = jnp.min(jnp.where(key == mkey, lane, _INT_MAX))

    rowmask = lane == i0
    x_col = jnp.sum(jnp.where(rowmask, hs, 0.0), axis=1, keepdims=True)
    y_col = jnp.sum(jnp.where(rowmask, ht, 0.0), axis=1, keepdims=True)
    h1 = (jnp.sum(x_col * W1a, axis=0, keepdims=True)
          + jnp.sum(y_col * W1b, axis=0, keepdims=True) + b1r[:, :])
    h1 = jnp.maximum(h1, 0.0)
    h2 = jnp.sum(W2T * h1, axis=1, keepdims=True) + b2c[:, :]
    h2 = jnp.maximum(h2, 0.0)
    pred0 = (jnp.sum(h2 * W3c[:, :]) + b3[0, 0]) / _K

    cond = (pred0 > 0.5) & ((jnp.abs(pred0 - 1.0) < sim) | (pred0 < sim))
    fw0 = jnp.where(member & cond, pred0, sim)
    S = jnp.sum(fw0)
    ws = fw0 / (S + 1e-6)
    m = jnp.max(ws)
    e = jnp.exp(ws - m)
    sm = e / jnp.sum(e)
    w = sm / (jnp.sum(sm) + 1e-6)
    sw = jnp.sum(w)

    scv = [jnp.sum(w * xs[c:c + 1, :]) for c in range(3)]
    tcv = [jnp.sum(w * xt[d:d + 1, :]) for d in range(3)]
    Hm = [[jnp.sum((w * xs[c:c + 1, :]) * xt[d:d + 1, :])
           - (2.0 - sw) * scv[c] * tcv[d] + (1e-6 if c == d else 0.0)
           for d in range(3)] for c in range(3)]

    A = [[sum(Hm[k][i] * Hm[k][j] for k in range(3)) for j in range(3)]
         for i in range(3)]
    V = [[jnp.float32(1.0 if i == j else 0.0) for j in range(3)]
         for i in range(3)]
    for _sweep in range(6):
        for (p, q) in ((0, 1), (0, 2), (1, 2)):
            apq = A[p][q]
            skip = jnp.abs(apq) < 1e-36
            tau = (A[q][q] - A[p][p]) / (2.0 * jnp.where(skip, 1.0, apq))
            sgn = jnp.where(tau >= 0.0, 1.0, -1.0)
            t = sgn / (jnp.abs(tau) + jnp.sqrt(1.0 + tau * tau))
            t = jnp.where(skip, 0.0, t)
            cc = 1.0 / jnp.sqrt(1.0 + t * t)
            ss = t * cc
            for k in range(3):
                akp, akq = A[k][p], A[k][q]
                A[k][p] = cc * akp - ss * akq
                A[k][q] = ss * akp + cc * akq
            for k in range(3):
                apk, aqk = A[p][k], A[q][k]
                A[p][k] = cc * apk - ss * aqk
                A[q][k] = ss * apk + cc * aqk
            for k in range(3):
                vkp, vkq = V[k][p], V[k][q]
                V[k][p] = cc * vkp - ss * vkq
                V[k][q] = ss * vkp + cc * vkq
    lam = [A[0][0], A[1][1], A[2][2]]
    cols = [[V[k][i] for k in range(3)] for i in range(3)]
    vdet = jnp.float32(1.0)
    for (i, j) in ((0, 1), (1, 2), (0, 1)):
        swp = lam[i] < lam[j]
        li, lj = lam[i], lam[j]
        lam[i] = jnp.where(swp, lj, li)
        lam[j] = jnp.where(swp, li, lj)
        vdet = jnp.where(swp, -vdet, vdet)
        for k in range(3):
            ci, cj = cols[i][k], cols[j][k]
            cols[i][k] = jnp.where(swp, cj, ci)
            cols[j][k] = jnp.where(swp, ci, cj)
    sig = [jnp.sqrt(jnp.maximum(l, 0.0)) for l in lam]
    Hv = [[sum(Hm[r][k] * cols[i][k] for k in range(3)) for r in range(3)]
          for i in range(2)]
    u1 = [Hv[0][r] / sig[0] for r in range(3)]
    n1 = jnp.sqrt(sum(u * u for u in u1))
    u1 = [u / n1 for u in u1]
    u2 = [Hv[1][r] / sig[1] for r in range(3)]
    n2 = jnp.sqrt(sum(u * u for u in u2))
    u2 = [u / n2 for u in u2]
    u3 = [(u1[1] * u2[2] - u1[2] * u2[1]) * vdet,
          (u1[2] * u2[0] - u1[0] * u2[2]) * vdet,
          (u1[0] * u2[1] - u1[1] * u2[0]) * vdet]
    Rm = [[cols[0][r] * u1[c] + cols[1][r] * u2[c] + cols[2][r] * u3[c]
           for c in range(3)] for r in range(3)]
    for r in range(3):
        for c in range(3):
            R_ref[r, c] = Rm[r][c]
        t_ref[r, 0] = tcv[r] - (Rm[r][0] * scv[0] + Rm[r][1] * scv[1]
                                + Rm[r][2] * scv[2])


def _vm(shape=None):
    return pl.BlockSpec(memory_space=pltpu.VMEM)


def _sm():
    return pl.BlockSpec(memory_space=pltpu.SMEM)


def kernel(h_src, x_src, edges_src, edge_attr_src, h_tgt, x_tgt, edges_tgt,
           edge_attr_tgt, corr, labels, gt_pose, W1, b1, W2, b2, W3, b3):
    f32 = jnp.float32
    hsT = jnp.transpose(h_src[0])
    htT = jnp.transpose(h_tgt[0])
    xT = jnp.transpose(
        jnp.concatenate([x_src[0], x_tgt[0]], axis=1))
    lab = labels.astype(f32)
    b1r = b1.reshape(1, _F)
    b2c = b2.reshape(_F // 2, 1)
    W3c = W3.reshape(_F // 2, 1)
    Rg = gt_pose[0, :3, :3]
    tg = gt_pose[0, :3, 3:4]
    b3m = b3.reshape(1, 1)

    loss, Rb, tb = pl.pallas_call(
        _main_kernel,
        out_shape=(
            jax.ShapeDtypeStruct((1, 1), f32),
            jax.ShapeDtypeStruct((3, 3), f32),
            jax.ShapeDtypeStruct((3, 1), f32),
        ),
        in_specs=[_vm(), _vm(), _vm(), _vm(),
                  _vm(), _vm(), _vm(), _vm(), _vm(),
                  _sm(), _sm(), _sm()],
        out_specs=(_sm(), _sm(), _sm()),
    )(hsT, htT, xT, lab, W1, b1r, W2, b2c, W3c, Rg, tg, b3m)

    return loss[0, 0], Rb[None], tb[:, 0][None]


# speedup vs baseline: 1.1765x; 1.0501x over previous
import jax
import jax.numpy as jnp
import numpy as np
from jax.experimental import pallas as pl
from jax.experimental.pallas import tpu as pltpu

_N = 8192
_F = 32
_K = 128
_INT_MIN = np.int32(-2147483648)
_INT_MAX = np.int32(2147483647)


def _main_kernel(hsT, htT, xT, lab, W1m, b1r, W2m, b2c, W3c,
                 gp, b3, loss_ref, R_ref, t_ref):
    hs = hsT[:, :]
    ht = htT[:, :]
    xs = xT[0:3, :]
    xt = xT[3:6, :]
    labv = lab[:, :]
    W1a = W1m[0:_F, :]
    W1b = W1m[_F:2 * _F, :]
    W2T = jnp.transpose(W2m[:, :])

    sim = jnp.sum(hs * ht, axis=0, keepdims=True)

    ch = jnp.zeros((1, _N), jnp.float32)
    for c in range(3):
        xtf_c = (gp[c, 0] * xs[0:1, :] + gp[c, 1] * xs[1:2, :]
                 + gp[c, 2] * xs[2:3, :] + gp[c, 3])
        d = xtf_c - xt[c:c + 1, :]
        ch = ch + d * d
    rot_loss = jnp.sum(ch * labv) / _N

    eps = 1e-8
    ns = jnp.sqrt(jnp.sum(hs * hs, axis=0, keepdims=True))
    nt = jnp.sqrt(jnp.sum(ht * ht, axis=0, keepdims=True))
    den = jnp.maximum(ns, eps) * jnp.maximum(nt, eps)
    fdiff = sim / den - labv
    feat_loss = jnp.sum(fdiff * fdiff) / _N
    loss_ref[0, 0] = rot_loss + feat_loss

    bits = jax.lax.bitcast_convert_type(sim, jnp.int32)
    key = bits ^ ((bits >> 31) & jnp.int32(0x7FFFFFFF))
    lane = jax.lax.broadcasted_iota(jnp.int32, (1, _N), 1)

    def bs_body(i, c):
        lo, hi = c
        floor_avg = (lo >> 1) + (hi >> 1) + (lo & hi & 1)
        mid = floor_avg + ((lo ^ hi) & 1)
        cnt = jnp.sum((key >= mid).astype(jnp.int32))
        take = cnt >= _K
        return (jnp.where(take, mid, lo),
                jnp.where(take, hi, mid - jnp.int32(1)))

    vK, _ = jax.lax.fori_loop(0, 34, bs_body, (_INT_MIN, _INT_MAX))

    gt_cnt = jnp.sum((key > vK).astype(jnp.int32))
    tslots = _K - gt_cnt
    tie = key == vK

    def ts_body(i, c):
        lo2, hi2 = c
        mid = (lo2 + hi2) // 2
        cnt = jnp.sum((tie & (lane <= mid)).astype(jnp.int32))
        take = cnt >= tslots
        return (jnp.where(take, lo2, mid + jnp.int32(1)),
                jnp.where(take, mid, hi2))

    tstar, _ = jax.lax.fori_loop(0, 14, ts_body,
                                 (jnp.int32(0), jnp.int32(_N - 1)))
    member = (key > vK) | (tie & (lane <= tstar))

    mkey = jnp.max(key)
    i0 = jnp.min(jnp.where(key == mkey, lane, _INT_MAX))

    rowmask = lane == i0
    x_col = jnp.sum(jnp.where(rowmask, hs, 0.0), axis=1, keepdims=True)
    y_col = jnp.sum(jnp.where(rowmask, ht, 0.0), axis=1, keepdims=True)
    h1 = (jnp.sum(x_col * W1a, axis=0, keepdims=True)
          + jnp.sum(y_col * W1b, axis=0, keepdims=True) + b1r[:, :])
    h1 = jnp.maximum(h1, 0.0)
    h2 = jnp.sum(W2T * h1, axis=1, keepdims=True) + b2c[:, :]
    h2 = jnp.maximum(h2, 0.0)
    pred0 = (jnp.sum(h2 * W3c[:, :]) + b3[0, 0]) / _K

    cond = (pred0 > 0.5) & ((jnp.abs(pred0 - 1.0) < sim) | (pred0 < sim))
    fw0 = jnp.where(member & cond, pred0, sim)
    S = jnp.sum(fw0)
    ws = fw0 / (S + 1e-6)
    m = jnp.max(ws)
    e = jnp.exp(ws - m)
    sm = e / jnp.sum(e)
    w = sm / (jnp.sum(sm) + 1e-6)
    sw = jnp.sum(w)

    scv = [jnp.sum(w * xs[c:c + 1, :]) for c in range(3)]
    tcv = [jnp.sum(w * xt[d:d + 1, :]) for d in range(3)]
    Hm = [[jnp.sum((w * xs[c:c + 1, :]) * xt[d:d + 1, :])
           - (2.0 - sw) * scv[c] * tcv[d] + (1e-6 if c == d else 0.0)
           for d in range(3)] for c in range(3)]

    A = [[sum(Hm[k][i] * Hm[k][j] for k in range(3)) for j in range(3)]
         for i in range(3)]
    V = [[jnp.float32(1.0 if i == j else 0.0) for j in range(3)]
         for i in range(3)]
    for _sweep in range(6):
        for (p, q) in ((0, 1), (0, 2), (1, 2)):
            apq = A[p][q]
            skip = jnp.abs(apq) < 1e-36
            tau = (A[q][q] - A[p][p]) / (2.0 * jnp.where(skip, 1.0, apq))
            sgn = jnp.where(tau >= 0.0, 1.0, -1.0)
            t = sgn / (jnp.abs(tau) + jnp.sqrt(1.0 + tau * tau))
            t = jnp.where(skip, 0.0, t)
            cc = 1.0 / jnp.sqrt(1.0 + t * t)
            ss = t * cc
            for k in range(3):
                akp, akq = A[k][p], A[k][q]
                A[k][p] = cc * akp - ss * akq
                A[k][q] = ss * akp + cc * akq
            for k in range(3):
                apk, aqk = A[p][k], A[q][k]
                A[p][k] = cc * apk - ss * aqk
                A[q][k] = ss * apk + cc * aqk
            for k in range(3):
                vkp, vkq = V[k][p], V[k][q]
                V[k][p] = cc * vkp - ss * vkq
                V[k][q] = ss * vkp + cc * vkq
    lam = [A[0][0], A[1][1], A[2][2]]
    cols = [[V[k][i] for k in range(3)] for i in range(3)]
    vdet = jnp.float32(1.0)
    for (i, j) in ((0, 1), (1, 2), (0, 1)):
        swp = lam[i] < lam[j]
        li, lj = lam[i], lam[j]
        lam[i] = jnp.where(swp, lj, li)
        lam[j] = jnp.where(swp, li, lj)
        vdet = jnp.where(swp, -vdet, vdet)
        for k in range(3):
            ci, cj = cols[i][k], cols[j][k]
            cols[i][k] = jnp.where(swp, cj, ci)
            cols[j][k] = jnp.where(swp, ci, cj)
    sig = [jnp.sqrt(jnp.maximum(l, 0.0)) for l in lam]
    Hv = [[sum(Hm[r][k] * cols[i][k] for k in range(3)) for r in range(3)]
          for i in range(2)]
    u1 = [Hv[0][r] / sig[0] for r in range(3)]
    n1 = jnp.sqrt(sum(u * u for u in u1))
    u1 = [u / n1 for u in u1]
    u2 = [Hv[1][r] / sig[1] for r in range(3)]
    n2 = jnp.sqrt(sum(u * u for u in u2))
    u2 = [u / n2 for u in u2]
    u3 = [(u1[1] * u2[2] - u1[2] * u2[1]) * vdet,
          (u1[2] * u2[0] - u1[0] * u2[2]) * vdet,
          (u1[0] * u2[1] - u1[1] * u2[0]) * vdet]
    Rm = [[cols[0][r] * u1[c] + cols[1][r] * u2[c] + cols[2][r] * u3[c]
           for c in range(3)] for r in range(3)]
    for r in range(3):
        for c in range(3):
            R_ref[r, c] = Rm[r][c]
        t_ref[r, 0] = tcv[r] - (Rm[r][0] * scv[0] + Rm[r][1] * scv[1]
                                + Rm[r][2] * scv[2])


def _vm(shape=None):
    return pl.BlockSpec(memory_space=pltpu.VMEM)


def _sm():
    return pl.BlockSpec(memory_space=pltpu.SMEM)


def kernel(h_src, x_src, edges_src, edge_attr_src, h_tgt, x_tgt, edges_tgt,
           edge_attr_tgt, corr, labels, gt_pose, W1, b1, W2, b2, W3, b3):
    f32 = jnp.float32
    hsT = jnp.transpose(h_src[0])
    htT = jnp.transpose(h_tgt[0])
    xT = jnp.transpose(
        jnp.concatenate([x_src[0], x_tgt[0]], axis=1))
    lab = labels.astype(f32)
    b1r = b1.reshape(1, _F)
    b2c = b2.reshape(_F // 2, 1)
    W3c = W3.reshape(_F // 2, 1)
    gp = gt_pose[0]
    b3m = b3.reshape(1, 1)

    loss, Rb, tb = pl.pallas_call(
        _main_kernel,
        out_shape=(
            jax.ShapeDtypeStruct((1, 1), f32),
            jax.ShapeDtypeStruct((3, 3), f32),
            jax.ShapeDtypeStruct((3, 1), f32),
        ),
        in_specs=[_vm(), _vm(), _vm(), _vm(),
                  _vm(), _vm(), _vm(), _vm(), _vm(),
                  _sm(), _sm()],
        out_specs=(_sm(), _sm(), _sm()),
    )(hsT, htT, xT, lab, W1, b1r, W2, b2c, W3c, gp, b3m)

    return loss[0, 0], Rb[None], tb[:, 0][None]
